# Initial kernel scaffold; baseline (speedup 1.0000x reference)
import jax
import jax.numpy as jnp
from jax.experimental import pallas as pl


def kernel(x, edge_index, batch, emb, W1, b1, W2, b2, p, Wd, bd, Wo, bo):
    raise NotImplementedError("write your pallas kernel here")


# speedup vs baseline: 17.4032x; 17.4032x over previous
import functools

import jax
import jax.numpy as jnp
from jax import lax
from jax.experimental import pallas as pl
from jax.experimental.pallas import tpu as pltpu
from jax.experimental.pallas import tpu_sc as plsc

N = 50000
E = 800000
G = 16
ED = 64
H = 32

NC, NS = 2, 16
NW = NC * NS
NCH = 98
NKC = 16
NPW = NCH * NKC
NPAD = NW * NPW
RPT = NPAD // NS

ECH = 128
EKC = 200
EW = ECH * EKC
EPAD = NW * EW
RB = 4
DCH = 40
SB = 40
NSUP = EKC // SB
IRND = SB // RB
ZCH = 56
ZR = RPT // ZCH

BR = 512
GRID = NPAD // BR

def _embed_deg_body(x2, dst2, emb, zvec, ones_h, h0, degp, xidx_v, rows_v,
                    dst_v, ones_v, zv, acc, gsem):
    c = lax.axis_index("c")
    s = lax.axis_index("s")
    wid = s * NC + c
    pltpu.sync_copy(zvec, zv)
    pltpu.sync_copy(zv, acc.at[pl.ds(s * RPT, RPT)])
    plsc.subcore_barrier()
    pltpu.sync_copy(x2.at[pl.ds(wid * NKC, NKC)], xidx_v)
    descs = []
    for k in range(NKC):
        descs.append(pltpu.async_copy(
            emb.at[xidx_v.at[k]], rows_v.at[pl.ds(k * NCH, NCH), :], gsem))
    pltpu.sync_copy(ones_h, ones_v)

    def deg_round(r, carry):
        pltpu.sync_copy(dst2.at[pl.ds(wid * EKC + r * DCH, DCH)], dst_v)
        for k in range(DCH):
            pltpu.sync_copy(ones_v, acc.at[dst_v.at[k]], add=True)
        return carry

    lax.fori_loop(0, EKC // DCH, deg_round, 0)
    for d in descs:
        d.wait()
    pltpu.sync_copy(rows_v, h0.at[pl.ds(wid * NPW, NPW), :])
    plsc.subcore_barrier()
    pltpu.sync_copy(acc.at[pl.ds(s * RPT, RPT)], zv)
    pltpu.sync_copy(zv, degp.at[c, s])


@functools.cache
def _build_embed_deg():
    mesh = plsc.VectorSubcoreMesh(core_axis_name="c", subcore_axis_name="s",
                                  num_cores=NC, num_subcores=NS)
    return pl.kernel(
        _embed_deg_body,
        out_type=(
            jax.ShapeDtypeStruct((NPAD, ED), jnp.float32),
            jax.ShapeDtypeStruct((NC, NS, RPT), jnp.float32),
        ),
        mesh=mesh,
        compiler_params=pltpu.CompilerParams(use_tc_tiling_on_sc=False),
        scratch_types=[
            pltpu.VMEM((NKC, NCH), jnp.int32),
            pltpu.VMEM((NPW, ED), jnp.float32),
            pltpu.VMEM((DCH, ECH), jnp.int32),
            pltpu.VMEM((ECH,), jnp.float32),
            pltpu.VMEM((RPT,), jnp.float32),
            pltpu.VMEM_SHARED((NPAD,), jnp.float32),
            pltpu.SemaphoreType.DMA,
        ],
    )


def _embed_deg(x2, dst2, emb, zvec, ones_h):
    return _build_embed_deg()(x2, dst2, emb, zvec, ones_h)


def _edge_scatter_body(src2, dst2, u, zrows, sp, sidx_v, didx_v, rows_v, zb,
                       acc, sem0, sem1, sem2, sem3):
    sems = [sem0, sem1, sem2, sem3]
    c = lax.axis_index("c")
    s = lax.axis_index("s")
    wid = s * NC + c
    pltpu.sync_copy(zrows, zb)

    def zrnd(j, carry):
        pltpu.sync_copy(zb, acc.at[pl.ds(s * RPT + j * ZCH, ZCH), :])
        return carry

    lax.fori_loop(0, ZR, zrnd, 0)
    plsc.subcore_barrier()

    def suprnd(ss, carry):
        base = wid * EKC + ss * SB
        pltpu.sync_copy(src2.at[pl.ds(base, SB)], sidx_v)
        pltpu.sync_copy(dst2.at[pl.ds(base, SB)], didx_v)
        for b in range(RB):
            pltpu.async_copy(u.at[sidx_v.at[b]], rows_v.at[b], sems[b])
        for k in range(SB):
            b = k % RB
            pltpu.make_async_copy(u.at[sidx_v.at[k]], rows_v.at[b],
                                  sems[b]).wait()
            pltpu.sync_copy(rows_v.at[b], acc.at[didx_v.at[k]], add=True)
            if k + RB < SB:
                pltpu.async_copy(u.at[sidx_v.at[k + RB]], rows_v.at[b],
                                 sems[b])
        return carry

    lax.fori_loop(0, NSUP, suprnd, 0)
    plsc.subcore_barrier()

    def wrnd(j, carry):
        pltpu.sync_copy(acc.at[pl.ds(s * RPT + j * ZCH, ZCH), :], zb)
        pltpu.sync_copy(zb, sp.at[c, pl.ds(s * RPT + j * ZCH, ZCH), :])
        return carry

    lax.fori_loop(0, ZR, wrnd, 0)


@functools.cache
def _build_edge_scatter():
    mesh = plsc.VectorSubcoreMesh(core_axis_name="c", subcore_axis_name="s",
                                  num_cores=NC, num_subcores=NS)
    return pl.kernel(
        _edge_scatter_body,
        out_type=jax.ShapeDtypeStruct((NC, NPAD, H), jnp.float32),
        mesh=mesh,
        compiler_params=pltpu.CompilerParams(use_tc_tiling_on_sc=False),
        scratch_types=[
            pltpu.VMEM((SB, ECH), jnp.int32),
            pltpu.VMEM((SB, ECH), jnp.int32),
            pltpu.VMEM((RB, ECH, H), jnp.float32),
            pltpu.VMEM((ZCH, H), jnp.float32),
            pltpu.VMEM_SHARED((NPAD, H), jnp.float32),
            pltpu.SemaphoreType.DMA,
            pltpu.SemaphoreType.DMA,
            pltpu.SemaphoreType.DMA,
            pltpu.SemaphoreType.DMA,
        ],
    )


def _edge_scatter(src2, dst2, u, zrows):
    return _build_edge_scatter()(src2, dst2, u, zrows)


def _dense1_body(h0_ref, degT_ref, w_ref, u_ref):
    deg = jnp.sum(degT_ref[...], axis=1, keepdims=True) + 1.0
    dinv = 1.0 / jnp.sqrt(deg)
    u_ref[...] = jnp.dot(h0_ref[...], w_ref[...],
                         preferred_element_type=jnp.float32,
                         precision=lax.Precision.HIGHEST) * dinv


_dense1 = pl.pallas_call(
    _dense1_body,
    grid=(GRID,),
    in_specs=[
        pl.BlockSpec((BR, ED), lambda i: (i, 0)),
        pl.BlockSpec((BR, NC), lambda i: (i, 0)),
        pl.BlockSpec((ED, H), lambda i: (0, 0)),
    ],
    out_specs=pl.BlockSpec((BR, H), lambda i: (i, 0)),
    out_shape=jax.ShapeDtypeStruct((NPAD, H), jnp.float32),
)


def _dense2_body(sp_ref, u1_ref, degT_ref, w_ref, b1_ref, u2_ref):
    deg = jnp.sum(degT_ref[...], axis=1, keepdims=True) + 1.0
    dinv = 1.0 / jnp.sqrt(deg)
    ssum = sp_ref[0] + sp_ref[1] + u1_ref[...]
    h1 = jnp.maximum(ssum * dinv + b1_ref[...], 0.0)
    u2_ref[...] = jnp.dot(h1, w_ref[...],
                          preferred_element_type=jnp.float32,
                         precision=lax.Precision.HIGHEST) * dinv


_dense2 = pl.pallas_call(
    _dense2_body,
    grid=(GRID,),
    in_specs=[
        pl.BlockSpec((NC, BR, H), lambda i: (0, i, 0)),
        pl.BlockSpec((BR, H), lambda i: (i, 0)),
        pl.BlockSpec((BR, NC), lambda i: (i, 0)),
        pl.BlockSpec((H, H), lambda i: (0, 0)),
        pl.BlockSpec((1, H), lambda i: (0, 0)),
    ],
    out_specs=pl.BlockSpec((BR, H), lambda i: (i, 0)),
    out_shape=jax.ShapeDtypeStruct((NPAD, H), jnp.float32),
)


def _score_body(sp_ref, u2_ref, degT_ref, b2_ref, p_ref, hs_ref, key_ref):
    deg = jnp.sum(degT_ref[...], axis=1, keepdims=True) + 1.0
    dinv = 1.0 / jnp.sqrt(deg)
    ssum = sp_ref[0] + sp_ref[1] + u2_ref[...]
    h2 = jnp.maximum(ssum * dinv + b2_ref[...], 0.0)
    pv = p_ref[...]
    inv_norm = lax.rsqrt(jnp.sum(pv * pv))
    z = jnp.sum(h2 * pv, axis=1, keepdims=True) * inv_norm
    sc = jnp.tanh(z)
    hs_ref[...] = h2 * sc
    ib = lax.bitcast_convert_type(sc, jnp.int32)
    key = jnp.where(ib < 0, jnp.int32(-2147483648) - ib, ib)
    key_ref[...] = jnp.reshape(key, (1, 1, BR))


_score = pl.pallas_call(
    _score_body,
    grid=(GRID,),
    in_specs=[
        pl.BlockSpec((NC, BR, H), lambda i: (0, i, 0)),
        pl.BlockSpec((BR, H), lambda i: (i, 0)),
        pl.BlockSpec((BR, NC), lambda i: (i, 0)),
        pl.BlockSpec((1, H), lambda i: (0, 0)),
        pl.BlockSpec((1, H), lambda i: (0, 0)),
    ],
    out_specs=[
        pl.BlockSpec((BR, H), lambda i: (i, 0)),
        pl.BlockSpec((1, 1, BR), lambda i: (i, 0, 0)),
    ],
    out_shape=[
        jax.ShapeDtypeStruct((NPAD, H), jnp.float32),
        jax.ShapeDtypeStruct((GRID, 1, BR), jnp.int32),
    ],
)


def _select_body(key_ref, batch_ref, keep_ref):
    key = key_ref[...]
    b = batch_ref[...]
    onehot = [b == g for g in range(G)]
    cnt = [jnp.sum(jnp.where(onehot[g], 1.0, 0.0)) for g in range(G)]
    kf = [jnp.floor((4.0 * cnt[g] + 4.0) / 5.0) for g in range(G)]

    def vstep(_, lohi):
        lo, hi = lohi
        nlo, nhi = [], []
        for g in range(G):
            mid = lo[g] + ((hi[g] - lo[g]) >> 1)
            c = jnp.sum(jnp.where(onehot[g] & (key >= mid), 1.0, 0.0))
            take = c >= kf[g]
            nlo.append(jnp.where(take, mid, lo[g]))
            nhi.append(jnp.where(take, hi[g], mid))
        return tuple(nlo), tuple(nhi)

    lo0 = tuple(jnp.int32(-1065353217) for _ in range(G))
    hi0 = tuple(jnp.int32(1065353218) for _ in range(G))
    v, _ = lax.fori_loop(0, 32, vstep, (lo0, hi0))

    riota = lax.broadcasted_iota(jnp.int32, (GRID, BR), 0)
    ciota = lax.broadcasted_iota(jnp.int32, (GRID, BR), 1)
    idxv = riota * BR + ciota
    tie = [onehot[g] & (key == v[g]) for g in range(G)]
    need = [kf[g] - jnp.sum(jnp.where(onehot[g] & (key > v[g]), 1.0, 0.0))
            for g in range(G)]

    def istep(_, lohi):
        lo, hi = lohi
        nlo, nhi = [], []
        for g in range(G):
            mid = lo[g] + ((hi[g] - lo[g]) >> 1)
            c = jnp.sum(jnp.where(tie[g] & (idxv <= mid), 1.0, 0.0))
            ok = c >= need[g]
            nlo.append(jnp.where(ok, lo[g], mid))
            nhi.append(jnp.where(ok, mid, hi[g]))
        return tuple(nlo), tuple(nhi)

    lo0i = tuple(jnp.int32(-1) for _ in range(G))
    hi0i = tuple(jnp.int32(NPAD) for _ in range(G))
    _, t = lax.fori_loop(0, 17, istep, (lo0i, hi0i))

    keep = jnp.zeros((GRID, BR), jnp.bool_)
    for g in range(G):
        keep = keep | (onehot[g] & ((key > v[g])
                                    | (tie[g] & (idxv <= t[g]))))
    keep_ref[...] = keep.astype(jnp.int32)


def _select(key2, batch2):
    return pl.pallas_call(
        _select_body,
        out_shape=jax.ShapeDtypeStruct((GRID, BR), jnp.int32),
    )(key2, batch2)


def _poolhead_body(hs_ref, keep_ref, batch_ref, wd_ref, bd_ref, wo_ref,
                   bo_ref, out_ref, acc_ref):
    i = pl.program_id(0)

    @pl.when(i == 0)
    def _():
        acc_ref[...] = jnp.full((G, H), -jnp.inf, jnp.float32)

    hs = hs_ref[...]
    b = jnp.reshape(batch_ref[...], (BR, 1))
    kp = jnp.reshape(keep_ref[...], (BR, 1)) > 0
    neg = jnp.float32(-jnp.inf)
    cols = []
    for g in range(G):
        m = kp & (b == g)
        cols.append(jnp.max(jnp.where(m, hs, neg), axis=0, keepdims=True))
    acc_ref[...] = jnp.maximum(acc_ref[...], jnp.concatenate(cols, axis=0))

    @pl.when(i == GRID - 1)
    def _():
        pooled = acc_ref[...]
        pooled = jnp.where(jnp.isfinite(pooled), pooled, 0.0)
        d = jnp.maximum(jnp.dot(pooled, wd_ref[...],
                                preferred_element_type=jnp.float32,
                         precision=lax.Precision.HIGHEST)
                        + bd_ref[...], 0.0)
        out_ref[...] = jnp.dot(d, wo_ref[...],
                               preferred_element_type=jnp.float32,
                         precision=lax.Precision.HIGHEST) \
            + bo_ref[...]


def _pool_head(hs, key3, batch2, Wd, bd, Wo, bo):
    keep2 = _select(key3.reshape(GRID, BR), batch2)
    keep3 = keep2.reshape(GRID, 1, BR)
    batch3 = batch2.reshape(GRID, 1, BR)
    cout = Wo.shape[1]
    return pl.pallas_call(
        _poolhead_body,
        grid=(GRID,),
        in_specs=[
            pl.BlockSpec((BR, H), lambda i: (i, 0)),
            pl.BlockSpec((1, 1, BR), lambda i: (i, 0, 0)),
            pl.BlockSpec((1, 1, BR), lambda i: (i, 0, 0)),
            pl.BlockSpec((H, Wd.shape[1]), lambda i: (0, 0)),
            pl.BlockSpec((1, Wd.shape[1]), lambda i: (0, 0)),
            pl.BlockSpec((Wo.shape[0], cout), lambda i: (0, 0)),
            pl.BlockSpec((1, cout), lambda i: (0, 0)),
        ],
        out_specs=pl.BlockSpec((G, cout), lambda i: (0, 0)),
        out_shape=jax.ShapeDtypeStruct((G, cout), jnp.float32),
        scratch_shapes=[pltpu.VMEM((G, H), jnp.float32)],
    )(hs, keep3, batch3, Wd, bd, Wo, bo)


def kernel(x, edge_index, batch, emb, W1, b1, W2, b2, p, Wd, bd, Wo, bo):
    xf = x[:, 0].astype(jnp.int32)
    x2 = jnp.concatenate(
        [xf, jnp.zeros((NPAD - N,), jnp.int32)]).reshape(NPAD // NCH, NCH)
    src = edge_index[0].astype(jnp.int32)
    dst = edge_index[1].astype(jnp.int32)
    src2 = jnp.concatenate(
        [src, jnp.zeros((EPAD - E,), jnp.int32)]).reshape(EPAD // ECH, ECH)
    dst2 = jnp.concatenate(
        [dst, jnp.full((EPAD - E,), N, jnp.int32)]).reshape(EPAD // ECH, ECH)
    batch2 = jnp.concatenate(
        [batch.astype(jnp.int32),
         jnp.full((NPAD - N,), G, jnp.int32)]).reshape(GRID, BR)
    zvec = jnp.zeros((RPT,), jnp.float32)
    zrows = jnp.zeros((ZCH, H), jnp.float32)
    ones_h = jnp.ones((ECH,), jnp.float32)

    h0, degp = _embed_deg(x2, dst2, emb, zvec, ones_h)
    degT = degp.reshape(NC, NPAD).T
    u1 = _dense1(h0, degT, W1)
    s1p = _edge_scatter(src2, dst2, u1, zrows)
    u2 = _dense2(s1p, u1, degT, W2, b1.reshape(1, H))
    s2p = _edge_scatter(src2, dst2, u2, zrows)
    hs, key2 = _score(s2p, u2, degT, b2.reshape(1, H), p.reshape(1, H))
    return _pool_head(hs, key2, batch2, Wd, bd.reshape(1, -1), Wo,
                      bo.reshape(1, -1))


# speedup vs baseline: 17.7568x; 1.0203x over previous
import functools

import jax
import jax.numpy as jnp
from jax import lax
from jax.experimental import pallas as pl
from jax.experimental.pallas import tpu as pltpu
from jax.experimental.pallas import tpu_sc as plsc

N = 50000
E = 800000
G = 16
ED = 64
H = 32

NC, NS = 2, 16
NW = NC * NS
NCH = 98
NKC = 16
NPW = NCH * NKC
NPAD = NW * NPW
RPT = NPAD // NS

ECH = 128
EKC = 200
EW = ECH * EKC
EPAD = NW * EW
RB = 4
DCH = 40
SB = 40
NSUP = EKC // SB
IRND = SB // RB
ZCH = 56
ZR = RPT // ZCH

BR = 512
GRID = NPAD // BR

def _embed_deg_body(x2, dst2, emb, zvec, ones_h, h0, degp, xidx_v, rows_v,
                    dst_v, ones_v, zv, acc, gsem):
    c = lax.axis_index("c")
    s = lax.axis_index("s")
    wid = s * NC + c
    pltpu.sync_copy(zvec, zv)
    pltpu.sync_copy(zv, acc.at[pl.ds(s * RPT, RPT)])
    plsc.subcore_barrier()
    pltpu.sync_copy(x2.at[pl.ds(wid * NKC, NKC)], xidx_v)
    descs = []
    for k in range(NKC):
        descs.append(pltpu.async_copy(
            emb.at[xidx_v.at[k]], rows_v.at[pl.ds(k * NCH, NCH), :], gsem))
    pltpu.sync_copy(ones_h, ones_v)

    def deg_round(r, carry):
        pltpu.sync_copy(dst2.at[pl.ds(wid * EKC + r * DCH, DCH)], dst_v)
        for k in range(DCH):
            pltpu.sync_copy(ones_v, acc.at[dst_v.at[k]], add=True)
        return carry

    lax.fori_loop(0, EKC // DCH, deg_round, 0)
    for d in descs:
        d.wait()
    pltpu.sync_copy(rows_v, h0.at[pl.ds(wid * NPW, NPW), :])
    plsc.subcore_barrier()
    pltpu.sync_copy(acc.at[pl.ds(s * RPT, RPT)], zv)
    pltpu.sync_copy(zv, degp.at[c, s])


@functools.cache
def _build_embed_deg():
    mesh = plsc.VectorSubcoreMesh(core_axis_name="c", subcore_axis_name="s",
                                  num_cores=NC, num_subcores=NS)
    return pl.kernel(
        _embed_deg_body,
        out_type=(
            jax.ShapeDtypeStruct((NPAD, ED), jnp.float32),
            jax.ShapeDtypeStruct((NC, NS, RPT), jnp.float32),
        ),
        mesh=mesh,
        compiler_params=pltpu.CompilerParams(use_tc_tiling_on_sc=False),
        scratch_types=[
            pltpu.VMEM((NKC, NCH), jnp.int32),
            pltpu.VMEM((NPW, ED), jnp.float32),
            pltpu.VMEM((DCH, ECH), jnp.int32),
            pltpu.VMEM((ECH,), jnp.float32),
            pltpu.VMEM((RPT,), jnp.float32),
            pltpu.VMEM_SHARED((NPAD,), jnp.float32),
            pltpu.SemaphoreType.DMA,
        ],
    )


def _embed_deg(x2, dst2, emb, zvec, ones_h):
    return _build_embed_deg()(x2, dst2, emb, zvec, ones_h)


def _edge_scatter_body(src2, dst2, u, zrows, sp, sidx_v, didx_v, rows_v, zb,
                       acc, gsem0, gsem1, gsem2, gsem3, ssem0, ssem1, ssem2,
                       ssem3):
    gsems = [gsem0, gsem1, gsem2, gsem3]
    ssems = [ssem0, ssem1, ssem2, ssem3]
    c = lax.axis_index("c")
    s = lax.axis_index("s")
    wid = s * NC + c
    pltpu.sync_copy(zrows, zb)
    zdescs = []
    for j in range(ZR):
        zdescs.append(pltpu.async_copy(
            zb, acc.at[pl.ds(s * RPT + j * ZCH, ZCH), :], gsem0))
    for d in zdescs:
        d.wait()
    plsc.subcore_barrier()

    def suprnd(ss, carry):
        base = wid * EKC + ss * SB
        pltpu.sync_copy(src2.at[pl.ds(base, SB)], sidx_v)
        pltpu.sync_copy(dst2.at[pl.ds(base, SB)], didx_v)
        LA = 2
        for j in range(LA):
            pltpu.async_copy(u.at[sidx_v.at[j]], rows_v.at[j % RB],
                             gsems[j % RB])
        for k in range(SB):
            b = k % RB
            j = k + LA
            if j < SB:
                bj = j % RB
                if j >= RB:
                    pltpu.make_async_copy(
                        rows_v.at[bj], acc.at[didx_v.at[j - RB]],
                        ssems[bj]).wait()
                pltpu.async_copy(u.at[sidx_v.at[j]], rows_v.at[bj],
                                 gsems[bj])
            pltpu.make_async_copy(u.at[sidx_v.at[k]], rows_v.at[b],
                                  gsems[b]).wait()
            pltpu.async_copy(rows_v.at[b], acc.at[didx_v.at[k]], ssems[b],
                             add=True)
        for k in range(SB - RB, SB):
            b = k % RB
            pltpu.make_async_copy(rows_v.at[b], acc.at[didx_v.at[k]],
                                  ssems[b]).wait()
        return carry

    lax.fori_loop(0, NSUP, suprnd, 0)
    plsc.subcore_barrier()
    WCH = 112
    WN = RPT // WCH
    for j in range(WN):
        b = j % RB
        if j >= RB:
            pltpu.make_async_copy(
                rows_v.at[b, pl.ds(0, WCH), :],
                sp.at[c, pl.ds(s * RPT + (j - RB) * WCH, WCH), :],
                ssems[b]).wait()
        pltpu.async_copy(acc.at[pl.ds(s * RPT + j * WCH, WCH), :],
                         rows_v.at[b, pl.ds(0, WCH), :], gsems[b])
        pltpu.make_async_copy(acc.at[pl.ds(s * RPT + j * WCH, WCH), :],
                              rows_v.at[b, pl.ds(0, WCH), :],
                              gsems[b]).wait()
        pltpu.async_copy(rows_v.at[b, pl.ds(0, WCH), :],
                         sp.at[c, pl.ds(s * RPT + j * WCH, WCH), :],
                         ssems[b])
    for j in range(WN - RB, WN):
        b = j % RB
        pltpu.make_async_copy(
            rows_v.at[b, pl.ds(0, WCH), :],
            sp.at[c, pl.ds(s * RPT + j * WCH, WCH), :], ssems[b]).wait()


@functools.cache
def _build_edge_scatter():
    mesh = plsc.VectorSubcoreMesh(core_axis_name="c", subcore_axis_name="s",
                                  num_cores=NC, num_subcores=NS)
    return pl.kernel(
        _edge_scatter_body,
        out_type=jax.ShapeDtypeStruct((NC, NPAD, H), jnp.float32),
        mesh=mesh,
        compiler_params=pltpu.CompilerParams(use_tc_tiling_on_sc=False),
        scratch_types=[
            pltpu.VMEM((SB, ECH), jnp.int32),
            pltpu.VMEM((SB, ECH), jnp.int32),
            pltpu.VMEM((RB, ECH, H), jnp.float32),
            pltpu.VMEM((ZCH, H), jnp.float32),
            pltpu.VMEM_SHARED((NPAD, H), jnp.float32),
            pltpu.SemaphoreType.DMA,
            pltpu.SemaphoreType.DMA,
            pltpu.SemaphoreType.DMA,
            pltpu.SemaphoreType.DMA,
            pltpu.SemaphoreType.DMA,
            pltpu.SemaphoreType.DMA,
            pltpu.SemaphoreType.DMA,
            pltpu.SemaphoreType.DMA,
        ],
    )


def _edge_scatter(src2, dst2, u, zrows):
    return _build_edge_scatter()(src2, dst2, u, zrows)


def _dense1_body(h0_ref, degT_ref, w_ref, u_ref):
    deg = jnp.sum(degT_ref[...], axis=1, keepdims=True) + 1.0
    dinv = 1.0 / jnp.sqrt(deg)
    u_ref[...] = jnp.dot(h0_ref[...], w_ref[...],
                         preferred_element_type=jnp.float32,
                         precision=lax.Precision.HIGHEST) * dinv


_dense1 = pl.pallas_call(
    _dense1_body,
    grid=(GRID,),
    in_specs=[
        pl.BlockSpec((BR, ED), lambda i: (i, 0)),
        pl.BlockSpec((BR, NC), lambda i: (i, 0)),
        pl.BlockSpec((ED, H), lambda i: (0, 0)),
    ],
    out_specs=pl.BlockSpec((BR, H), lambda i: (i, 0)),
    out_shape=jax.ShapeDtypeStruct((NPAD, H), jnp.float32),
)


def _dense2_body(sp_ref, u1_ref, degT_ref, w_ref, b1_ref, u2_ref):
    deg = jnp.sum(degT_ref[...], axis=1, keepdims=True) + 1.0
    dinv = 1.0 / jnp.sqrt(deg)
    ssum = sp_ref[0] + sp_ref[1] + u1_ref[...]
    h1 = jnp.maximum(ssum * dinv + b1_ref[...], 0.0)
    u2_ref[...] = jnp.dot(h1, w_ref[...],
                          preferred_element_type=jnp.float32,
                         precision=lax.Precision.HIGHEST) * dinv


_dense2 = pl.pallas_call(
    _dense2_body,
    grid=(GRID,),
    in_specs=[
        pl.BlockSpec((NC, BR, H), lambda i: (0, i, 0)),
        pl.BlockSpec((BR, H), lambda i: (i, 0)),
        pl.BlockSpec((BR, NC), lambda i: (i, 0)),
        pl.BlockSpec((H, H), lambda i: (0, 0)),
        pl.BlockSpec((1, H), lambda i: (0, 0)),
    ],
    out_specs=pl.BlockSpec((BR, H), lambda i: (i, 0)),
    out_shape=jax.ShapeDtypeStruct((NPAD, H), jnp.float32),
)


def _score_body(sp_ref, u2_ref, degT_ref, b2_ref, p_ref, hs_ref, key_ref):
    deg = jnp.sum(degT_ref[...], axis=1, keepdims=True) + 1.0
    dinv = 1.0 / jnp.sqrt(deg)
    ssum = sp_ref[0] + sp_ref[1] + u2_ref[...]
    h2 = jnp.maximum(ssum * dinv + b2_ref[...], 0.0)
    pv = p_ref[...]
    inv_norm = lax.rsqrt(jnp.sum(pv * pv))
    z = jnp.sum(h2 * pv, axis=1, keepdims=True) * inv_norm
    sc = jnp.tanh(z)
    hs_ref[...] = h2 * sc
    ib = lax.bitcast_convert_type(sc, jnp.int32)
    key = jnp.where(ib < 0, jnp.int32(-2147483648) - ib, ib)
    key_ref[...] = jnp.reshape(key, (1, 1, BR))


_score = pl.pallas_call(
    _score_body,
    grid=(GRID,),
    in_specs=[
        pl.BlockSpec((NC, BR, H), lambda i: (0, i, 0)),
        pl.BlockSpec((BR, H), lambda i: (i, 0)),
        pl.BlockSpec((BR, NC), lambda i: (i, 0)),
        pl.BlockSpec((1, H), lambda i: (0, 0)),
        pl.BlockSpec((1, H), lambda i: (0, 0)),
    ],
    out_specs=[
        pl.BlockSpec((BR, H), lambda i: (i, 0)),
        pl.BlockSpec((1, 1, BR), lambda i: (i, 0, 0)),
    ],
    out_shape=[
        jax.ShapeDtypeStruct((NPAD, H), jnp.float32),
        jax.ShapeDtypeStruct((GRID, 1, BR), jnp.int32),
    ],
)


def _select_body(key_ref, batch_ref, keep_ref):
    key = key_ref[...]
    b = batch_ref[...]
    onehot = [b == g for g in range(G)]
    cnt = [jnp.sum(jnp.where(onehot[g], 1.0, 0.0)) for g in range(G)]
    kf = [jnp.floor((4.0 * cnt[g] + 4.0) / 5.0) for g in range(G)]

    def vstep(_, lohi):
        lo, hi = lohi
        nlo, nhi = [], []
        for g in range(G):
            mid = lo[g] + ((hi[g] - lo[g]) >> 1)
            c = jnp.sum(jnp.where(onehot[g] & (key >= mid), 1.0, 0.0))
            take = c >= kf[g]
            nlo.append(jnp.where(take, mid, lo[g]))
            nhi.append(jnp.where(take, hi[g], mid))
        return tuple(nlo), tuple(nhi)

    lo0 = tuple(jnp.int32(-1065353217) for _ in range(G))
    hi0 = tuple(jnp.int32(1065353218) for _ in range(G))
    v, _ = lax.fori_loop(0, 32, vstep, (lo0, hi0))

    riota = lax.broadcasted_iota(jnp.int32, (GRID, BR), 0)
    ciota = lax.broadcasted_iota(jnp.int32, (GRID, BR), 1)
    idxv = riota * BR + ciota
    tie = [onehot[g] & (key == v[g]) for g in range(G)]
    need = [kf[g] - jnp.sum(jnp.where(onehot[g] & (key > v[g]), 1.0, 0.0))
            for g in range(G)]

    def istep(_, lohi):
        lo, hi = lohi
        nlo, nhi = [], []
        for g in range(G):
            mid = lo[g] + ((hi[g] - lo[g]) >> 1)
            c = jnp.sum(jnp.where(tie[g] & (idxv <= mid), 1.0, 0.0))
            ok = c >= need[g]
            nlo.append(jnp.where(ok, lo[g], mid))
            nhi.append(jnp.where(ok, mid, hi[g]))
        return tuple(nlo), tuple(nhi)

    lo0i = tuple(jnp.int32(-1) for _ in range(G))
    hi0i = tuple(jnp.int32(NPAD) for _ in range(G))
    _, t = lax.fori_loop(0, 17, istep, (lo0i, hi0i))

    keep = jnp.zeros((GRID, BR), jnp.bool_)
    for g in range(G):
        keep = keep | (onehot[g] & ((key > v[g])
                                    | (tie[g] & (idxv <= t[g]))))
    keep_ref[...] = keep.astype(jnp.int32)


def _select(key2, batch2):
    return pl.pallas_call(
        _select_body,
        out_shape=jax.ShapeDtypeStruct((GRID, BR), jnp.int32),
    )(key2, batch2)


def _poolhead_body(hs_ref, keep_ref, batch_ref, wd_ref, bd_ref, wo_ref,
                   bo_ref, out_ref, acc_ref):
    i = pl.program_id(0)

    @pl.when(i == 0)
    def _():
        acc_ref[...] = jnp.full((G, H), -jnp.inf, jnp.float32)

    hs = hs_ref[...]
    b = jnp.reshape(batch_ref[...], (BR, 1))
    kp = jnp.reshape(keep_ref[...], (BR, 1)) > 0
    neg = jnp.float32(-jnp.inf)
    cols = []
    for g in range(G):
        m = kp & (b == g)
        cols.append(jnp.max(jnp.where(m, hs, neg), axis=0, keepdims=True))
    acc_ref[...] = jnp.maximum(acc_ref[...], jnp.concatenate(cols, axis=0))

    @pl.when(i == GRID - 1)
    def _():
        pooled = acc_ref[...]
        pooled = jnp.where(jnp.isfinite(pooled), pooled, 0.0)
        d = jnp.maximum(jnp.dot(pooled, wd_ref[...],
                                preferred_element_type=jnp.float32,
                         precision=lax.Precision.HIGHEST)
                        + bd_ref[...], 0.0)
        out_ref[...] = jnp.dot(d, wo_ref[...],
                               preferred_element_type=jnp.float32,
                         precision=lax.Precision.HIGHEST) \
            + bo_ref[...]


def _pool_head(hs, key3, batch2, Wd, bd, Wo, bo):
    keep2 = _select(key3.reshape(GRID, BR), batch2)
    keep3 = keep2.reshape(GRID, 1, BR)
    batch3 = batch2.reshape(GRID, 1, BR)
    cout = Wo.shape[1]
    return pl.pallas_call(
        _poolhead_body,
        grid=(GRID,),
        in_specs=[
            pl.BlockSpec((BR, H), lambda i: (i, 0)),
            pl.BlockSpec((1, 1, BR), lambda i: (i, 0, 0)),
            pl.BlockSpec((1, 1, BR), lambda i: (i, 0, 0)),
            pl.BlockSpec((H, Wd.shape[1]), lambda i: (0, 0)),
            pl.BlockSpec((1, Wd.shape[1]), lambda i: (0, 0)),
            pl.BlockSpec((Wo.shape[0], cout), lambda i: (0, 0)),
            pl.BlockSpec((1, cout), lambda i: (0, 0)),
        ],
        out_specs=pl.BlockSpec((G, cout), lambda i: (0, 0)),
        out_shape=jax.ShapeDtypeStruct((G, cout), jnp.float32),
        scratch_shapes=[pltpu.VMEM((G, H), jnp.float32)],
    )(hs, keep3, batch3, Wd, bd, Wo, bo)


def kernel(x, edge_index, batch, emb, W1, b1, W2, b2, p, Wd, bd, Wo, bo):
    xf = x[:, 0].astype(jnp.int32)
    x2 = jnp.concatenate(
        [xf, jnp.zeros((NPAD - N,), jnp.int32)]).reshape(NPAD // NCH, NCH)
    src = edge_index[0].astype(jnp.int32)
    dst = edge_index[1].astype(jnp.int32)
    src2 = jnp.concatenate(
        [src, jnp.zeros((EPAD - E,), jnp.int32)]).reshape(EPAD // ECH, ECH)
    dst2 = jnp.concatenate(
        [dst, jnp.full((EPAD - E,), N, jnp.int32)]).reshape(EPAD // ECH, ECH)
    batch2 = jnp.concatenate(
        [batch.astype(jnp.int32),
         jnp.full((NPAD - N,), G, jnp.int32)]).reshape(GRID, BR)
    zvec = jnp.zeros((RPT,), jnp.float32)
    zrows = jnp.zeros((ZCH, H), jnp.float32)
    ones_h = jnp.ones((ECH,), jnp.float32)

    h0, degp = _embed_deg(x2, dst2, emb, zvec, ones_h)
    degT = degp.reshape(NC, NPAD).T
    u1 = _dense1(h0, degT, W1)
    s1p = _edge_scatter(src2, dst2, u1, zrows)
    u2 = _dense2(s1p, u1, degT, W2, b1.reshape(1, H))
    s2p = _edge_scatter(src2, dst2, u2, zrows)
    hs, key2 = _score(s2p, u2, degT, b2.reshape(1, H), p.reshape(1, H))
    return _pool_head(hs, key2, batch2, Wd, bd.reshape(1, -1), Wo,
                      bo.reshape(1, -1))


# speedup vs baseline: 18.8941x; 1.0640x over previous
import functools

import jax
import jax.numpy as jnp
from jax import lax
from jax.experimental import pallas as pl
from jax.experimental.pallas import tpu as pltpu
from jax.experimental.pallas import tpu_sc as plsc

N = 50000
E = 800000
G = 16
ED = 64
H = 32

NC, NS = 2, 16
NW = NC * NS
NCH = 98
NKC = 16
NPW = NCH * NKC
NPAD = NW * NPW
RPT = NPAD // NS

ECH = 128
EKC = 200
EW = ECH * EKC
EPAD = NW * EW
RB = 4
DCH = 40
SB = 40
NSUP = EKC // SB
IRND = SB // RB
ZCH = 56
ZR = RPT // ZCH

BR = 512
GRID = NPAD // BR

def _embed_deg_body(x2, dst2, emb, zvec, ones_h, h0, degp, xidx_v, rows_v,
                    dst_v, ones_v, zv, acc, gsem):
    c = lax.axis_index("c")
    s = lax.axis_index("s")
    wid = s * NC + c
    pltpu.sync_copy(zvec, zv)
    pltpu.sync_copy(zv, acc.at[pl.ds(s * RPT, RPT)])
    plsc.subcore_barrier()
    pltpu.sync_copy(x2.at[pl.ds(wid * NKC, NKC)], xidx_v)
    descs = []
    for k in range(NKC):
        descs.append(pltpu.async_copy(
            emb.at[xidx_v.at[k]], rows_v.at[pl.ds(k * NCH, NCH), :], gsem))
    pltpu.sync_copy(ones_h, ones_v)

    def deg_round(r, carry):
        pltpu.sync_copy(dst2.at[pl.ds(wid * EKC + r * DCH, DCH)], dst_v)
        for k in range(DCH):
            pltpu.sync_copy(ones_v, acc.at[dst_v.at[k]], add=True)
        return carry

    lax.fori_loop(0, EKC // DCH, deg_round, 0)
    for d in descs:
        d.wait()
    pltpu.sync_copy(rows_v, h0.at[pl.ds(wid * NPW, NPW), :])
    plsc.subcore_barrier()
    pltpu.sync_copy(acc.at[pl.ds(s * RPT, RPT)], zv)
    pltpu.sync_copy(zv, degp.at[c, s])


@functools.cache
def _build_embed_deg():
    mesh = plsc.VectorSubcoreMesh(core_axis_name="c", subcore_axis_name="s",
                                  num_cores=NC, num_subcores=NS)
    return pl.kernel(
        _embed_deg_body,
        out_type=(
            jax.ShapeDtypeStruct((NPAD, ED), jnp.float32),
            jax.ShapeDtypeStruct((NC, NS, RPT), jnp.float32),
        ),
        mesh=mesh,
        compiler_params=pltpu.CompilerParams(use_tc_tiling_on_sc=False),
        scratch_types=[
            pltpu.VMEM((NKC, NCH), jnp.int32),
            pltpu.VMEM((NPW, ED), jnp.float32),
            pltpu.VMEM((DCH, ECH), jnp.int32),
            pltpu.VMEM((ECH,), jnp.float32),
            pltpu.VMEM((RPT,), jnp.float32),
            pltpu.VMEM_SHARED((NPAD,), jnp.float32),
            pltpu.SemaphoreType.DMA,
        ],
    )


def _embed_deg(x2, dst2, emb, zvec, ones_h):
    return _build_embed_deg()(x2, dst2, emb, zvec, ones_h)


def _edge_scatter_body(src2, dst2, u, zrows, sp, sidx_v, didx_v, rows_v, zb,
                       acc, gsem0, gsem1, gsem2, gsem3, ssem0, ssem1, ssem2,
                       ssem3):
    gsems = [gsem0, gsem1, gsem2, gsem3]
    ssems = [ssem0, ssem1, ssem2, ssem3]
    c = lax.axis_index("c")
    s = lax.axis_index("s")
    wid = s * NC + c
    pltpu.sync_copy(zrows, zb)
    zdescs = []
    for j in range(ZR):
        zdescs.append(pltpu.async_copy(
            zb, acc.at[pl.ds(s * RPT + j * ZCH, ZCH), :], gsem0))
    for d in zdescs:
        d.wait()
    plsc.subcore_barrier()

    def suprnd(ss, carry):
        base = wid * EKC + ss * SB
        pltpu.sync_copy(src2.at[pl.ds(base, SB)], sidx_v)
        pltpu.sync_copy(dst2.at[pl.ds(base, SB)], didx_v)
        LA = 2
        for j in range(LA):
            pltpu.async_copy(u.at[sidx_v.at[j]], rows_v.at[j % RB],
                             gsems[j % RB])
        for k in range(SB):
            b = k % RB
            j = k + LA
            if j < SB:
                bj = j % RB
                if j >= RB:
                    pltpu.make_async_copy(
                        rows_v.at[bj], acc.at[didx_v.at[j - RB]],
                        ssems[bj]).wait()
                pltpu.async_copy(u.at[sidx_v.at[j]], rows_v.at[bj],
                                 gsems[bj])
            pltpu.make_async_copy(u.at[sidx_v.at[k]], rows_v.at[b],
                                  gsems[b]).wait()
            pltpu.async_copy(rows_v.at[b], acc.at[didx_v.at[k]], ssems[b],
                             add=True)
        for k in range(SB - RB, SB):
            b = k % RB
            pltpu.make_async_copy(rows_v.at[b], acc.at[didx_v.at[k]],
                                  ssems[b]).wait()
        return carry

    lax.fori_loop(0, NSUP, suprnd, 0)
    plsc.subcore_barrier()
    WCH = 112
    WN = RPT // WCH
    for j in range(WN):
        b = j % RB
        if j >= RB:
            pltpu.make_async_copy(
                rows_v.at[b, pl.ds(0, WCH), :],
                sp.at[c, pl.ds(s * RPT + (j - RB) * WCH, WCH), :],
                ssems[b]).wait()
        pltpu.async_copy(acc.at[pl.ds(s * RPT + j * WCH, WCH), :],
                         rows_v.at[b, pl.ds(0, WCH), :], gsems[b])
        pltpu.make_async_copy(acc.at[pl.ds(s * RPT + j * WCH, WCH), :],
                              rows_v.at[b, pl.ds(0, WCH), :],
                              gsems[b]).wait()
        pltpu.async_copy(rows_v.at[b, pl.ds(0, WCH), :],
                         sp.at[c, pl.ds(s * RPT + j * WCH, WCH), :],
                         ssems[b])
    for j in range(WN - RB, WN):
        b = j % RB
        pltpu.make_async_copy(
            rows_v.at[b, pl.ds(0, WCH), :],
            sp.at[c, pl.ds(s * RPT + j * WCH, WCH), :], ssems[b]).wait()


@functools.cache
def _build_edge_scatter():
    mesh = plsc.VectorSubcoreMesh(core_axis_name="c", subcore_axis_name="s",
                                  num_cores=NC, num_subcores=NS)
    return pl.kernel(
        _edge_scatter_body,
        out_type=jax.ShapeDtypeStruct((NC, NPAD, H), jnp.float32),
        mesh=mesh,
        compiler_params=pltpu.CompilerParams(use_tc_tiling_on_sc=False),
        scratch_types=[
            pltpu.VMEM((SB, ECH), jnp.int32),
            pltpu.VMEM((SB, ECH), jnp.int32),
            pltpu.VMEM((RB, ECH, H), jnp.float32),
            pltpu.VMEM((ZCH, H), jnp.float32),
            pltpu.VMEM_SHARED((NPAD, H), jnp.float32),
            pltpu.SemaphoreType.DMA,
            pltpu.SemaphoreType.DMA,
            pltpu.SemaphoreType.DMA,
            pltpu.SemaphoreType.DMA,
            pltpu.SemaphoreType.DMA,
            pltpu.SemaphoreType.DMA,
            pltpu.SemaphoreType.DMA,
            pltpu.SemaphoreType.DMA,
        ],
    )


def _edge_scatter(src2, dst2, u, zrows):
    return _build_edge_scatter()(src2, dst2, u, zrows)


def _dense1_body(h0_ref, degT_ref, w_ref, u_ref):
    deg = jnp.sum(degT_ref[...], axis=1, keepdims=True) + 1.0
    dinv = 1.0 / jnp.sqrt(deg)
    u_ref[...] = jnp.dot(h0_ref[...], w_ref[...],
                         preferred_element_type=jnp.float32) * dinv


_dense1 = pl.pallas_call(
    _dense1_body,
    grid=(GRID,),
    in_specs=[
        pl.BlockSpec((BR, ED), lambda i: (i, 0)),
        pl.BlockSpec((BR, NC), lambda i: (i, 0)),
        pl.BlockSpec((ED, H), lambda i: (0, 0)),
    ],
    out_specs=pl.BlockSpec((BR, H), lambda i: (i, 0)),
    out_shape=jax.ShapeDtypeStruct((NPAD, H), jnp.float32),
)


def _dense2_body(sp_ref, u1_ref, degT_ref, w_ref, b1_ref, u2_ref):
    deg = jnp.sum(degT_ref[...], axis=1, keepdims=True) + 1.0
    dinv = 1.0 / jnp.sqrt(deg)
    ssum = sp_ref[0] + sp_ref[1] + u1_ref[...]
    h1 = jnp.maximum(ssum * dinv + b1_ref[...], 0.0)
    u2_ref[...] = jnp.dot(h1, w_ref[...],
                          preferred_element_type=jnp.float32) * dinv


_dense2 = pl.pallas_call(
    _dense2_body,
    grid=(GRID,),
    in_specs=[
        pl.BlockSpec((NC, BR, H), lambda i: (0, i, 0)),
        pl.BlockSpec((BR, H), lambda i: (i, 0)),
        pl.BlockSpec((BR, NC), lambda i: (i, 0)),
        pl.BlockSpec((H, H), lambda i: (0, 0)),
        pl.BlockSpec((1, H), lambda i: (0, 0)),
    ],
    out_specs=pl.BlockSpec((BR, H), lambda i: (i, 0)),
    out_shape=jax.ShapeDtypeStruct((NPAD, H), jnp.float32),
)


def _score_body(sp_ref, u2_ref, degT_ref, b2_ref, p_ref, hs_ref, key_ref):
    deg = jnp.sum(degT_ref[...], axis=1, keepdims=True) + 1.0
    dinv = 1.0 / jnp.sqrt(deg)
    ssum = sp_ref[0] + sp_ref[1] + u2_ref[...]
    h2 = jnp.maximum(ssum * dinv + b2_ref[...], 0.0)
    pv = p_ref[...]
    z = jnp.sum(h2 * pv, axis=1, keepdims=True) / jnp.sqrt(jnp.sum(pv * pv))
    sc = jnp.tanh(z)
    hs_ref[...] = h2 * sc
    ib = lax.bitcast_convert_type(sc, jnp.int32)
    key = jnp.where(ib < 0, jnp.int32(-2147483648) - ib, ib)
    key_ref[...] = jnp.reshape(key, (1, 1, BR))


_score = pl.pallas_call(
    _score_body,
    grid=(GRID,),
    in_specs=[
        pl.BlockSpec((NC, BR, H), lambda i: (0, i, 0)),
        pl.BlockSpec((BR, H), lambda i: (i, 0)),
        pl.BlockSpec((BR, NC), lambda i: (i, 0)),
        pl.BlockSpec((1, H), lambda i: (0, 0)),
        pl.BlockSpec((1, H), lambda i: (0, 0)),
    ],
    out_specs=[
        pl.BlockSpec((BR, H), lambda i: (i, 0)),
        pl.BlockSpec((1, 1, BR), lambda i: (i, 0, 0)),
    ],
    out_shape=[
        jax.ShapeDtypeStruct((NPAD, H), jnp.float32),
        jax.ShapeDtypeStruct((GRID, 1, BR), jnp.int32),
    ],
)


def _select_compute(key, b):
    onehot = [b == g for g in range(G)]
    cnt = [jnp.sum(jnp.where(onehot[g], 1.0, 0.0)) for g in range(G)]
    kf = [jnp.floor((4.0 * cnt[g] + 4.0) / 5.0) for g in range(G)]

    def vstep(_, lohi):
        lo, hi = lohi
        nlo, nhi = [], []
        for g in range(G):
            mid = lo[g] + ((hi[g] - lo[g]) >> 1)
            c = jnp.sum(jnp.where(onehot[g] & (key >= mid), 1.0, 0.0))
            take = c >= kf[g]
            nlo.append(jnp.where(take, mid, lo[g]))
            nhi.append(jnp.where(take, hi[g], mid))
        return tuple(nlo), tuple(nhi)

    lo0 = tuple(jnp.int32(-1065353217) for _ in range(G))
    hi0 = tuple(jnp.int32(1065353218) for _ in range(G))

    def vcond(lohi):
        lo, hi = lohi
        live = hi[0] - lo[0]
        for g in range(1, G):
            live = jnp.maximum(live, hi[g] - lo[g])
        return live > 1

    v, _ = lax.while_loop(vcond, lambda lh: vstep(0, lh), (lo0, hi0))

    riota = lax.broadcasted_iota(jnp.int32, (GRID, BR), 0)
    ciota = lax.broadcasted_iota(jnp.int32, (GRID, BR), 1)
    idxv = riota * BR + ciota
    tie = [onehot[g] & (key == v[g]) for g in range(G)]
    need = [kf[g] - jnp.sum(jnp.where(onehot[g] & (key > v[g]), 1.0, 0.0))
            for g in range(G)]

    def istep(_, lohi):
        lo, hi = lohi
        nlo, nhi = [], []
        for g in range(G):
            mid = lo[g] + ((hi[g] - lo[g]) >> 1)
            c = jnp.sum(jnp.where(tie[g] & (idxv <= mid), 1.0, 0.0))
            ok = c >= need[g]
            nlo.append(jnp.where(ok, lo[g], mid))
            nhi.append(jnp.where(ok, mid, hi[g]))
        return tuple(nlo), tuple(nhi)

    lo0i = tuple(jnp.int32(-1) for _ in range(G))
    hi0i = tuple(jnp.int32(NPAD) for _ in range(G))

    def icond(lohi):
        lo, hi = lohi
        live = hi[0] - lo[0]
        for g in range(1, G):
            live = jnp.maximum(live, hi[g] - lo[g])
        return live > 1

    _, t = lax.while_loop(icond, lambda lh: istep(0, lh), (lo0i, hi0i))

    keep = jnp.zeros((GRID, BR), jnp.bool_)
    for g in range(G):
        keep = keep | (onehot[g] & ((key > v[g])
                                    | (tie[g] & (idxv <= t[g]))))
    return keep.astype(jnp.int32)


def _poolhead_body(hs_ref, key_ref, batch_ref, wd_ref, bd_ref, wo_ref,
                   bo_ref, out_ref, acc_ref, keep_scr):
    i = pl.program_id(0)

    @pl.when(i == 0)
    def _():
        acc_ref[...] = jnp.full((G, H), -jnp.inf, jnp.float32)
        keep_scr[...] = _select_compute(key_ref[...], batch_ref[...])

    hs = hs_ref[...]
    b = jnp.reshape(batch_ref[pl.ds(i, 1), :], (BR, 1))
    kp = jnp.reshape(keep_scr[pl.ds(i, 1), :], (BR, 1)) > 0
    neg = jnp.float32(-jnp.inf)
    cols = []
    for g in range(G):
        m = kp & (b == g)
        cols.append(jnp.max(jnp.where(m, hs, neg), axis=0, keepdims=True))
    acc_ref[...] = jnp.maximum(acc_ref[...], jnp.concatenate(cols, axis=0))

    @pl.when(i == GRID - 1)
    def _():
        pooled = acc_ref[...]
        pooled = jnp.where(jnp.isfinite(pooled), pooled, 0.0)
        d = jnp.maximum(jnp.dot(pooled, wd_ref[...],
                                preferred_element_type=jnp.float32)
                        + bd_ref[...], 0.0)
        out_ref[...] = jnp.dot(d, wo_ref[...],
                               preferred_element_type=jnp.float32) \
            + bo_ref[...]


def _pool_head(hs, key3, batch2, Wd, bd, Wo, bo):
    key2 = key3.reshape(GRID, BR)
    cout = Wo.shape[1]
    return pl.pallas_call(
        _poolhead_body,
        grid=(GRID,),
        in_specs=[
            pl.BlockSpec((BR, H), lambda i: (i, 0)),
            pl.BlockSpec((GRID, BR), lambda i: (0, 0)),
            pl.BlockSpec((GRID, BR), lambda i: (0, 0)),
            pl.BlockSpec((H, Wd.shape[1]), lambda i: (0, 0)),
            pl.BlockSpec((1, Wd.shape[1]), lambda i: (0, 0)),
            pl.BlockSpec((Wo.shape[0], cout), lambda i: (0, 0)),
            pl.BlockSpec((1, cout), lambda i: (0, 0)),
        ],
        out_specs=pl.BlockSpec((G, cout), lambda i: (0, 0)),
        out_shape=jax.ShapeDtypeStruct((G, cout), jnp.float32),
        scratch_shapes=[pltpu.VMEM((G, H), jnp.float32),
                        pltpu.VMEM((GRID, BR), jnp.int32)],
    )(hs, key2, batch2, Wd, bd, Wo, bo)


def kernel(x, edge_index, batch, emb, W1, b1, W2, b2, p, Wd, bd, Wo, bo):
    xf = x[:, 0].astype(jnp.int32)
    x2 = jnp.concatenate(
        [xf, jnp.zeros((NPAD - N,), jnp.int32)]).reshape(NPAD // NCH, NCH)
    src = edge_index[0].astype(jnp.int32)
    dst = edge_index[1].astype(jnp.int32)
    src2 = jnp.concatenate(
        [src, jnp.zeros((EPAD - E,), jnp.int32)]).reshape(EPAD // ECH, ECH)
    dst2 = jnp.concatenate(
        [dst, jnp.full((EPAD - E,), N, jnp.int32)]).reshape(EPAD // ECH, ECH)
    batch2 = jnp.concatenate(
        [batch.astype(jnp.int32),
         jnp.full((NPAD - N,), G, jnp.int32)]).reshape(GRID, BR)
    zvec = jnp.zeros((RPT,), jnp.float32)
    zrows = jnp.zeros((ZCH, H), jnp.float32)
    ones_h = jnp.ones((ECH,), jnp.float32)

    h0, degp = _embed_deg(x2, dst2, emb, zvec, ones_h)
    degT = degp.reshape(NC, NPAD).T
    u1 = _dense1(h0, degT, W1)
    s1p = _edge_scatter(src2, dst2, u1, zrows)
    u2 = _dense2(s1p, u1, degT, W2, b1.reshape(1, H))
    s2p = _edge_scatter(src2, dst2, u2, zrows)
    hs, key2 = _score(s2p, u2, degT, b2.reshape(1, H), p.reshape(1, H))
    return _pool_head(hs, key2, batch2, Wd, bd.reshape(1, -1), Wo,
                      bo.reshape(1, -1))
